# baseline (speedup 1.0000x reference)
import functools

import jax
import jax.numpy as jnp
from jax import lax
from jax.experimental import pallas as pl
from jax.experimental.pallas import tpu as pltpu
from jax.experimental.pallas import tpu_sc as plsc

_N, _E, _D, _G = 10000, 320000, 128, 64
_HD = _D // 2
_NC, _NS = 2, 16
_EPS = _E // _NS
_ECH = 125
_NCHUNK = _EPS // _ECH
_NP = 10240
_RPS = _NP // _NS
_RCH = 128
_RNCH = _RPS // _RCH


_NB = 4
_PF = 3


def _sc_body(feat_h, src_h, dst_h, zero_h, out_h,
             src_v, dst_v, big, acc,
             gs0, gs1, gs2, gs3):
  gsem = (gs0, gs1, gs2, gs3)
  ssem = gsem
  c = lax.axis_index("c")
  s = lax.axis_index("s")
  row0 = s * _RPS
  gw = tuple(big.at[pl.ds(b * _RCH, _ECH)] for b in range(_NB))

  pltpu.sync_copy(src_h.at[c, s], src_v)
  pltpu.sync_copy(dst_h.at[s], dst_v)

  bigr = big.at[pl.ds(0, _RPS)]
  pltpu.sync_copy(zero_h, bigr)
  pltpu.sync_copy(bigr, acc.at[pl.ds(row0, _RPS)])
  plsc.subcore_barrier()

  for b in range(_PF):
    pltpu.async_copy(feat_h.at[src_v.at[b]], gw[b], gsem[b])

  def step(i, carry):
    for b in range(_NB):
      j = _NB * i + b
      pltpu.make_async_copy(feat_h.at[src_v.at[j]], gw[b], gsem[b]).wait()
      pltpu.async_copy(gw[b], acc.at[dst_v.at[j]], ssem[b], add=True)
      jn = j + _PF
      bn = (b + _PF) % _NB

      @pl.when(jn < _NCHUNK)
      def _(jn=jn, bn=bn):
        @pl.when(jn >= _NB)
        def _():
          pltpu.make_async_copy(
              gw[bn], acc.at[dst_v.at[jn - _NB]], ssem[bn]).wait()

        pltpu.async_copy(feat_h.at[src_v.at[jn]], gw[bn], gsem[bn])

    return carry

  lax.fori_loop(0, _NCHUNK // _NB, step, 0)

  for b in range(_NB):
    j = _NCHUNK - _NB + b
    pltpu.make_async_copy(gw[b], acc.at[dst_v.at[j]], ssem[b]).wait()
  plsc.subcore_barrier()

  pltpu.sync_copy(acc.at[pl.ds(row0, _RPS)], bigr)
  pltpu.sync_copy(bigr, out_h.at[c, pl.ds(row0, _RPS)])


@functools.cache
def _sc_seg_sum_fn():
  return pl.kernel(
      _sc_body,
      out_type=jax.ShapeDtypeStruct((_NC, _NP, _HD), jnp.float32),
      mesh=plsc.VectorSubcoreMesh(
          core_axis_name="c", subcore_axis_name="s",
          num_cores=_NC, num_subcores=_NS),
      compiler_params=pltpu.CompilerParams(use_tc_tiling_on_sc=False),
      scratch_types=[
          pltpu.VMEM((_NCHUNK, _ECH), jnp.int32),
          pltpu.VMEM((_NCHUNK, _ECH), jnp.int32),
          pltpu.VMEM((max(_NB * _RCH, _RPS), _HD), jnp.float32),
          pltpu.VMEM_SHARED((_NP, _HD), jnp.float32),
      ] + [pltpu.SemaphoreType.DMA] * _NB,
  )


def _sc_seg_sum(feat_stacked, src2, dst, zeros_chunk):
  return _sc_seg_sum_fn()(feat_stacked, src2, dst, zeros_chunk)


def _gin_mlp(z, W1, b1, g, be, W2, b2):
  h = jnp.dot(z, W1, preferred_element_type=jnp.float32) + b1
  mean = jnp.mean(h, axis=0, keepdims=True)
  var = jnp.mean((h - mean) ** 2, axis=0, keepdims=True)
  h = (h - mean) / jnp.sqrt(var + 1e-5) * g + be
  h = jnp.maximum(h, 0.0)
  h = jnp.dot(h, W2, preferred_element_type=jnp.float32) + b2
  return jnp.maximum(h, 0.0)


def _tc1_body(x, p, W1, b1, g1, be1, W2, b2, h1_out):
  z = x[...] + jnp.concatenate([p[0, :_N], p[1, :_N]], axis=1)
  h1_out[...] = _gin_mlp(z, W1[...], b1[...], g1[...], be1[...],
                         W2[...], b2[...])


def _tc2_body(h1, q, W3, b3, g2, be2, W4, b4, Wr1, br1, Wr2, br2, Wr3, br3,
              Wm1, bm1, Wm2, bm2, batch2d, out_o, xrec_o):
  z = h1[...] + jnp.concatenate([q[0, :_N], q[1, :_N]], axis=1)
  h2 = _gin_mlp(z, W3[...], b3[...], g2[...], be2[...], W4[...], b4[...])

  r = jnp.maximum(jnp.dot(h2, Wr1[...],
                          preferred_element_type=jnp.float32) + br1[...], 0.0)
  r = jnp.maximum(jnp.dot(r, Wr2[...],
                          preferred_element_type=jnp.float32) + br2[...], 0.0)
  xrec_o[...] = jnp.maximum(
      jnp.dot(r, Wr3[...], preferred_element_type=jnp.float32) + br3[...], 0.0)

  gids = lax.broadcasted_iota(jnp.int32, (_N, _G), 1)
  onehot = (batch2d[...] == gids).astype(jnp.float32)
  pooled = lax.dot_general(onehot, h2, (((0,), (0,)), ((), ())),
                           preferred_element_type=jnp.float32)
  m = jnp.maximum(jnp.dot(pooled, Wm1[...],
                          preferred_element_type=jnp.float32) + bm1[...], 0.0)
  out_o[...] = jnp.dot(m, Wm2[...],
                       preferred_element_type=jnp.float32) + bm2[...]


_tc1 = pl.pallas_call(
    _tc1_body,
    out_shape=jax.ShapeDtypeStruct((_N, _D), jnp.float32),
)

_tc2 = pl.pallas_call(
    _tc2_body,
    out_shape=(
        jax.ShapeDtypeStruct((_G, 64), jnp.float32),
        jax.ShapeDtypeStruct((_N, 4), jnp.float32),
    ),
)


def kernel(x, W1, b1, g1, be1, W2, b2, W3, b3, g2, be2, W4, b4,
           Wr1, br1, Wr2, br2, Wr3, br3, Wm1, bm1, Wm2, bm2,
           edge_index, batch):
  src = edge_index[0].reshape(_NS, _NCHUNK, _ECH)
  src2 = jnp.stack([2 * src, 2 * src + 1])
  dst = edge_index[1].reshape(_NS, _NCHUNK, _ECH)
  zeros_chunk = jnp.zeros((_RPS, _HD), jnp.float32)

  p1 = _sc_seg_sum(x.reshape(2 * _N, _HD), src2, dst, zeros_chunk)
  h1 = _tc1(x, p1, W1, b1, g1, be1, W2, b2)
  q = _sc_seg_sum(h1.reshape(2 * _N, _HD), src2, dst, zeros_chunk)
  out, x_rec = _tc2(h1, q, W3, b3, g2, be2, W4, b4,
                    Wr1, br1, Wr2, br2, Wr3, br3,
                    Wm1, bm1, Wm2, bm2,
                    batch.reshape(_N, 1))
  return (out, x_rec)


# speedup vs baseline: 1.0759x; 1.0759x over previous
import functools

import jax
import jax.numpy as jnp
from jax import lax
from jax.experimental import pallas as pl
from jax.experimental.pallas import tpu as pltpu
from jax.experimental.pallas import tpu_sc as plsc

_N, _E, _D, _G = 10000, 320000, 128, 64
_HD = _D // 2
_NC, _NS = 2, 16
_EPS = _E // _NS
_ECH = 125
_NCHUNK = _EPS // _ECH
_NP = 10240
_RPS = _NP // _NS
_RCH = 128
_RNCH = _RPS // _RCH


_NB = 4
_PF = 3


def _sc_body(feat_h, src_h, dst_h, zero_h, out_h,
             src_v, dst_v, big, acc,
             gs0, gs1, gs2, gs3):
  gsem = (gs0, gs1, gs2, gs3)
  ssem = gsem
  c = lax.axis_index("c")
  s = lax.axis_index("s")
  row0 = s * _RPS
  gw = tuple(big.at[pl.ds(b * _RCH, _ECH)] for b in range(_NB))

  pltpu.sync_copy(src_h.at[c, s], src_v)
  pltpu.sync_copy(dst_h.at[s], dst_v)

  bigr = big.at[pl.ds(0, _RPS)]
  pltpu.sync_copy(zero_h, bigr)
  pltpu.sync_copy(bigr, acc.at[pl.ds(row0, _RPS)])
  plsc.subcore_barrier()

  for b in range(_PF):
    pltpu.async_copy(feat_h.at[src_v.at[b]], gw[b], gsem[b])

  def step(i, carry):
    for b in range(_NB):
      j = _NB * i + b
      pltpu.make_async_copy(feat_h.at[src_v.at[j]], gw[b], gsem[b]).wait()
      pltpu.async_copy(gw[b], acc.at[dst_v.at[j]], ssem[b], add=True)
      jn = j + _PF
      bn = (b + _PF) % _NB

      @pl.when(jn < _NCHUNK)
      def _(jn=jn, bn=bn):
        @pl.when(jn >= _NB)
        def _():
          pltpu.make_async_copy(
              gw[bn], acc.at[dst_v.at[jn - _NB]], ssem[bn]).wait()

        pltpu.async_copy(feat_h.at[src_v.at[jn]], gw[bn], gsem[bn])

    return carry

  lax.fori_loop(0, _NCHUNK // _NB, step, 0)

  for b in range(_NB):
    j = _NCHUNK - _NB + b
    pltpu.make_async_copy(gw[b], acc.at[dst_v.at[j]], ssem[b]).wait()
  plsc.subcore_barrier()

  pltpu.sync_copy(acc.at[pl.ds(row0, _RPS)], bigr)
  pltpu.sync_copy(bigr, out_h.at[c, pl.ds(row0, _RPS)])


@functools.cache
def _sc_seg_sum_fn():
  return pl.kernel(
      _sc_body,
      out_type=jax.ShapeDtypeStruct((_NC, _NP, _HD), jnp.float32),
      mesh=plsc.VectorSubcoreMesh(
          core_axis_name="c", subcore_axis_name="s",
          num_cores=_NC, num_subcores=_NS),
      compiler_params=pltpu.CompilerParams(use_tc_tiling_on_sc=False),
      scratch_types=[
          pltpu.VMEM((_NCHUNK, _ECH), jnp.int32),
          pltpu.VMEM((_NCHUNK, _ECH), jnp.int32),
          pltpu.VMEM((max(_NB * _RCH, _RPS), _HD), jnp.float32),
          pltpu.VMEM_SHARED((_NP, _HD), jnp.float32),
      ] + [pltpu.SemaphoreType.DMA] * _NB,
  )


def _sc_seg_sum(feat_stacked, src2, dst, zeros_chunk):
  return _sc_seg_sum_fn()(feat_stacked, src2, dst, zeros_chunk)


def _gin_mlp(z, W1, b1, g, be, W2, b2):
  h = jnp.dot(z, W1, preferred_element_type=jnp.float32) + b1
  mean = jnp.mean(h, axis=0, keepdims=True)
  var = jnp.mean((h - mean) ** 2, axis=0, keepdims=True)
  h = (h - mean) / jnp.sqrt(var + 1e-5) * g + be
  h = jnp.maximum(h, 0.0)
  h = jnp.dot(h, W2, preferred_element_type=jnp.float32) + b2
  return jnp.maximum(h, 0.0)


def _unstack(q):
  P, Q = q[0], q[1]
  a = jnp.concatenate(
      [P[:, :_HD], Q[:, :_HD], P[:, _HD:], Q[:, _HD:]], axis=1)
  return a.reshape(_NP, _D)[:_N]


def _tc1_body(x, p, W1, b1, g1, be1, W2, b2, h1_out):
  z = x[...] + _unstack(p)
  h1_out[...] = _gin_mlp(z, W1[...], b1[...], g1[...], be1[...],
                         W2[...], b2[...])


def _tc2_body(h1, q, W3, b3, g2, be2, W4, b4, Wr1, br1, Wr2, br2, Wr3, br3,
              Wm1, bm1, Wm2, bm2, batch2d, out_o, xrec_o):
  z = h1[...] + _unstack(q)
  h2 = _gin_mlp(z, W3[...], b3[...], g2[...], be2[...], W4[...], b4[...])

  r = jnp.maximum(jnp.dot(h2, Wr1[...],
                          preferred_element_type=jnp.float32) + br1[...], 0.0)
  r = jnp.maximum(jnp.dot(r, Wr2[...],
                          preferred_element_type=jnp.float32) + br2[...], 0.0)
  xrec_o[...] = jnp.maximum(
      jnp.dot(r, Wr3[...], preferred_element_type=jnp.float32) + br3[...], 0.0)

  gids = lax.broadcasted_iota(jnp.int32, (_N, _G), 1)
  onehot = (batch2d[...] == gids).astype(jnp.float32)
  pooled = lax.dot_general(onehot, h2, (((0,), (0,)), ((), ())),
                           preferred_element_type=jnp.float32)
  m = jnp.maximum(jnp.dot(pooled, Wm1[...],
                          preferred_element_type=jnp.float32) + bm1[...], 0.0)
  out_o[...] = jnp.dot(m, Wm2[...],
                       preferred_element_type=jnp.float32) + bm2[...]


_tc1 = pl.pallas_call(
    _tc1_body,
    out_shape=jax.ShapeDtypeStruct((_N, _D), jnp.float32),
)

_tc2 = pl.pallas_call(
    _tc2_body,
    out_shape=(
        jax.ShapeDtypeStruct((_G, 64), jnp.float32),
        jax.ShapeDtypeStruct((_N, 4), jnp.float32),
    ),
)


def kernel(x, W1, b1, g1, be1, W2, b2, W3, b3, g2, be2, W4, b4,
           Wr1, br1, Wr2, br2, Wr3, br3, Wm1, bm1, Wm2, bm2,
           edge_index, batch):
  src = edge_index[0].reshape(_NS, _NCHUNK, _ECH)
  src2 = jnp.stack([2 * src, 2 * src + 1])
  dst = edge_index[1].reshape(_NS, _NCHUNK, _ECH)
  zeros_chunk = jnp.zeros((_RPS, _HD), jnp.float32)

  pview = lambda t: t.reshape(_NC, _NP // 2, _D)
  p1 = _sc_seg_sum(x.reshape(2 * _N, _HD), src2, dst, zeros_chunk)
  h1 = _tc1(x, pview(p1), W1, b1, g1, be1, W2, b2)
  q = _sc_seg_sum(h1.reshape(2 * _N, _HD), src2, dst, zeros_chunk)
  out, x_rec = _tc2(h1, pview(q), W3, b3, g2, be2, W4, b4,
                    Wr1, br1, Wr2, br2, Wr3, br3,
                    Wm1, bm1, Wm2, bm2,
                    batch.reshape(_N, 1))
  return (out, x_rec)


# speedup vs baseline: 1.0922x; 1.0151x over previous
import functools

import jax
import jax.numpy as jnp
from jax import lax
from jax.experimental import pallas as pl
from jax.experimental.pallas import tpu as pltpu
from jax.experimental.pallas import tpu_sc as plsc

_N, _E, _D, _G = 10000, 320000, 128, 64
_HD = _D // 2
_NC, _NS = 2, 16
_EPS = _E // _NS
_ECH = 125
_NCHUNK = _EPS // _ECH
_NP = 10240
_RPS = _NP // _NS
_RCH = 128
_RNCH = _RPS // _RCH


_NB = 4
_PF = 3


def _sc_body(feat_h, src_h, dst_h, zero_h, out_h,
             src_v, dst_v, big, acc,
             gs0, gs1, gs2, gs3):
  gsem = (gs0, gs1, gs2, gs3)
  ssem = gsem
  c = lax.axis_index("c")
  s = lax.axis_index("s")
  row0 = s * _RPS
  gw = tuple(big.at[pl.ds(b * _RCH, _ECH)] for b in range(_NB))

  pltpu.sync_copy(src_h.at[c, s], src_v)
  pltpu.sync_copy(dst_h.at[s], dst_v)

  pltpu.sync_copy(zero_h, acc.at[pl.ds(row0, _RPS)])
  plsc.subcore_barrier()

  for b in range(_PF):
    pltpu.async_copy(feat_h.at[src_v.at[b]], gw[b], gsem[b])

  def step(i, carry):
    for b in range(_NB):
      j = _NB * i + b
      pltpu.make_async_copy(feat_h.at[src_v.at[j]], gw[b], gsem[b]).wait()
      pltpu.async_copy(gw[b], acc.at[dst_v.at[j]], ssem[b], add=True)
      jn = j + _PF
      bn = (b + _PF) % _NB

      @pl.when(jn < _NCHUNK)
      def _(jn=jn, bn=bn):
        @pl.when(jn >= _NB)
        def _():
          pltpu.make_async_copy(
              gw[bn], acc.at[dst_v.at[jn - _NB]], ssem[bn]).wait()

        pltpu.async_copy(feat_h.at[src_v.at[jn]], gw[bn], gsem[bn])

    return carry

  lax.fori_loop(0, _NCHUNK // _NB, step, 0)

  for b in range(_NB):
    j = _NCHUNK - _NB + b
    pltpu.make_async_copy(gw[b], acc.at[dst_v.at[j]], ssem[b]).wait()
  plsc.subcore_barrier()

  pltpu.sync_copy(acc.at[pl.ds(row0, _RPS)], out_h.at[c, pl.ds(row0, _RPS)])


@functools.cache
def _sc_seg_sum_fn():
  return pl.kernel(
      _sc_body,
      out_type=jax.ShapeDtypeStruct((_NC, _NP, _HD), jnp.float32),
      mesh=plsc.VectorSubcoreMesh(
          core_axis_name="c", subcore_axis_name="s",
          num_cores=_NC, num_subcores=_NS),
      compiler_params=pltpu.CompilerParams(use_tc_tiling_on_sc=False),
      scratch_types=[
          pltpu.VMEM((_NCHUNK, _ECH), jnp.int32),
          pltpu.VMEM((_NCHUNK, _ECH), jnp.int32),
          pltpu.VMEM((max(_NB * _RCH, _RPS), _HD), jnp.float32),
          pltpu.VMEM_SHARED((_NP, _HD), jnp.float32),
      ] + [pltpu.SemaphoreType.DMA] * _NB,
  )


def _sc_seg_sum(feat_stacked, src2, dst, zeros_chunk):
  return _sc_seg_sum_fn()(feat_stacked, src2, dst, zeros_chunk)


def _gin_mlp(z, W1, b1, g, be, W2, b2):
  h = jnp.dot(z, W1, preferred_element_type=jnp.float32) + b1
  mean = jnp.mean(h, axis=0, keepdims=True)
  var = jnp.mean((h - mean) ** 2, axis=0, keepdims=True)
  h = (h - mean) / jnp.sqrt(var + 1e-5) * g + be
  h = jnp.maximum(h, 0.0)
  h = jnp.dot(h, W2, preferred_element_type=jnp.float32) + b2
  return jnp.maximum(h, 0.0)


def _unstack(q):
  P, Q = q[0], q[1]
  a = jnp.concatenate(
      [P[:, :_HD], Q[:, :_HD], P[:, _HD:], Q[:, _HD:]], axis=1)
  return a.reshape(_NP, _D)[:_N]


def _tc1_body(x, p, W1, b1, g1, be1, W2, b2, h1_out):
  z = x[...] + _unstack(p)
  h1_out[...] = _gin_mlp(z, W1[...], b1[...], g1[...], be1[...],
                         W2[...], b2[...])


def _tc2_body(h1, q, W3, b3, g2, be2, W4, b4, Wr1, br1, Wr2, br2, Wr3, br3,
              Wm1, bm1, Wm2, bm2, batch2d, out_o, xrec_o):
  z = h1[...] + _unstack(q)
  h2 = _gin_mlp(z, W3[...], b3[...], g2[...], be2[...], W4[...], b4[...])

  r = jnp.maximum(jnp.dot(h2, Wr1[...],
                          preferred_element_type=jnp.float32) + br1[...], 0.0)
  r = jnp.maximum(jnp.dot(r, Wr2[...],
                          preferred_element_type=jnp.float32) + br2[...], 0.0)
  xrec_o[...] = jnp.maximum(
      jnp.dot(r, Wr3[...], preferred_element_type=jnp.float32) + br3[...], 0.0)

  gids = lax.broadcasted_iota(jnp.int32, (_N, _G), 1)
  onehot = (batch2d[...] == gids).astype(jnp.float32)
  pooled = lax.dot_general(onehot, h2, (((0,), (0,)), ((), ())),
                           preferred_element_type=jnp.float32)
  m = jnp.maximum(jnp.dot(pooled, Wm1[...],
                          preferred_element_type=jnp.float32) + bm1[...], 0.0)
  out_o[...] = jnp.dot(m, Wm2[...],
                       preferred_element_type=jnp.float32) + bm2[...]


_tc1 = pl.pallas_call(
    _tc1_body,
    out_shape=jax.ShapeDtypeStruct((_N, _D), jnp.float32),
)

_tc2 = pl.pallas_call(
    _tc2_body,
    out_shape=(
        jax.ShapeDtypeStruct((_G, 64), jnp.float32),
        jax.ShapeDtypeStruct((_N, 4), jnp.float32),
    ),
)


def kernel(x, W1, b1, g1, be1, W2, b2, W3, b3, g2, be2, W4, b4,
           Wr1, br1, Wr2, br2, Wr3, br3, Wm1, bm1, Wm2, bm2,
           edge_index, batch):
  src = edge_index[0].reshape(_NS, _NCHUNK, _ECH)
  src2 = jnp.stack([2 * src, 2 * src + 1])
  dst = edge_index[1].reshape(_NS, _NCHUNK, _ECH)
  zeros_chunk = jnp.zeros((_RPS, _HD), jnp.float32)

  pview = lambda t: t.reshape(_NC, _NP // 2, _D)
  p1 = _sc_seg_sum(x.reshape(2 * _N, _HD), src2, dst, zeros_chunk)
  h1 = _tc1(x, pview(p1), W1, b1, g1, be1, W2, b2)
  q = _sc_seg_sum(h1.reshape(2 * _N, _HD), src2, dst, zeros_chunk)
  out, x_rec = _tc2(h1, pview(q), W3, b3, g2, be2, W4, b4,
                    Wr1, br1, Wr2, br2, Wr3, br3,
                    Wm1, bm1, Wm2, bm2,
                    batch.reshape(_N, 1))
  return (out, x_rec)
